# baseline (speedup 1.0000x reference)
import functools

import jax
import jax.numpy as jnp
from jax import lax
from jax.experimental import pallas as pl
from jax.experimental.pallas import tpu as pltpu
from jax.experimental.pallas import tpu_sc as plsc

N = 10240
E = 327680
NUMROI = 128
HID = 64
IN_HEAD = 4
F = IN_HEAD * HID
G = 80
PER_G = 128

LN = 16
NSUB = 16
HALF = 128
ROW = HALF + LN
CHUNK = 40
NSLOT = 4
EPT = E // NSUB
NCHUNK = EPT // CHUNK
RPT = N // NSUB

_GDN = lax.GatherDimensionNumbers(
    offset_dims=(), collapsed_slice_dims=(0,), start_index_map=(0,))


def _lane_gather(vec, idx):
    return lax.gather(vec, idx[:, None], _GDN, (1,),
                      mode=lax.GatherScatterMode.PROMISE_IN_BOUNDS)


def _make_edge_kernel(heads, hpc, chph):
    mesh = plsc.VectorSubcoreMesh(core_axis_name="c", subcore_axis_name="s",
                                  num_cores=2, num_subcores=NSUB)

    scratch = []
    for _ in range(NSLOT):
        scratch += [
            pltpu.VMEM((CHUNK,), jnp.int32),
            pltpu.VMEM((CHUNK,), jnp.int32),
            pltpu.VMEM((CHUNK,), jnp.int32),
            pltpu.VMEM((CHUNK, ROW), jnp.float32),
            pltpu.VMEM((CHUNK, LN), jnp.float32),
            pltpu.VMEM((CHUNK, LN), jnp.float32),
            pltpu.SemaphoreType.DMA,
            pltpu.SemaphoreType.DMA,
        ]
    scratch.append(pltpu.VMEM_SHARED((N, ROW), jnp.float32))

    @functools.partial(
        pl.kernel, mesh=mesh,
        compiler_params=pltpu.CompilerParams(use_tc_tiling_on_sc=False),
        out_type=jax.ShapeDtypeStruct((2, N, ROW), jnp.float32),
        scratch_types=scratch,
    )
    def edge_kernel(hh, smx, gsrc, dstp, sct, zrow, out, *scr):
        slots = [scr[8 * b:8 * b + 8] for b in range(NSLOT)]
        accum = scr[8 * NSLOT]
        c = lax.axis_index("c")
        s = lax.axis_index("s")
        r0 = s * RPT
        pltpu.sync_copy(zrow.at[pl.ds(r0, RPT)], accum.at[pl.ds(r0, RPT)])
        plsc.subcore_barrier()

        iota = lax.iota(jnp.int32, LN)
        lmask = iota < hpc
        hb = (c * hpc) % heads
        idx_a = jnp.where(lmask, iota + hb, 0)
        idx_b = jnp.where(lmask, iota + heads + hb, 0)
        base_lim = s * EPT + (NCHUNK - 1) * CHUNK

        def start_chunk(gc, slot):
            gidx, didx, scidx, hbuf, smsrc, smdst, sem_g, _ = slot
            base = jnp.minimum(s * EPT + gc * CHUNK, base_lim)
            pltpu.sync_copy(gsrc.at[c, pl.ds(base, CHUNK)], gidx)
            pltpu.sync_copy(dstp.at[pl.ds(base, CHUNK)], didx)
            pltpu.sync_copy(sct.at[pl.ds(base, CHUNK)], scidx)
            pltpu.async_copy(hh.at[gidx], hbuf, sem_g)
            pltpu.async_copy(smx.at[gidx], smsrc, sem_g)
            pltpu.async_copy(smx.at[didx], smdst, sem_g)

        def wait_gathers(slot):
            gidx, didx, _, hbuf, smsrc, smdst, sem_g, _ = slot
            pltpu.make_async_copy(hh.at[gidx], hbuf, sem_g).wait()
            pltpu.make_async_copy(smx.at[gidx], smsrc, sem_g).wait()
            pltpu.make_async_copy(smx.at[didx], smdst, sem_g).wait()

        def compute_chunk(slot):
            _, _, _, hbuf, smsrc, smdst, _, _ = slot

            def edge_body(k4, carry2):
                for u in range(4):
                    k = k4 * 4 + u
                    sv = smsrc[k]
                    dv = smdst[k]
                    e = _lane_gather(sv, idx_a) + _lane_gather(dv, idx_b)
                    e = jnp.where(lmask, e, 0.0)
                    e = jnp.where(e >= 0.0, e, 0.2 * e)
                    w = jnp.exp(e)
                    hbuf[k, pl.ds(HALF, LN)] = jnp.where(lmask, w, 0.0)
                    ws = [_lane_gather(w, jnp.full((LN,), h, jnp.int32))
                          for h in range(hpc)]
                    for j in range(HALF // LN):
                        hl = (j * LN) // chph
                        hbuf[k, pl.ds(j * LN, LN)] = (
                            hbuf[k, pl.ds(j * LN, LN)] * ws[hl])
                return carry2

            lax.fori_loop(0, CHUNK // 4, edge_body, 0)

        def issue_scatter(slot):
            _, _, scidx, hbuf, _, _, _, sem_s = slot
            pltpu.async_copy(hbuf, accum.at[scidx], sem_s, add=True)

        def wait_scatter(slot):
            _, _, scidx, hbuf, _, _, _, sem_s = slot
            pltpu.make_async_copy(hbuf, accum.at[scidx], sem_s).wait()

        for g in range(2):
            start_chunk(g, slots[g])

        def pipe_body(i, carry):
            for b in range(NSLOT):
                gc = i * NSLOT + b
                slot = slots[b]
                wait_gathers(slot)
                compute_chunk(slot)
                issue_scatter(slot)
                s2 = (b + 2) % NSLOT
                if b < 2:
                    @pl.when(i > 0)
                    def _():
                        wait_scatter(slots[s2])
                else:
                    wait_scatter(slots[s2])
                start_chunk(gc + 2, slots[s2])
            return carry

        lax.fori_loop(0, NCHUNK // NSLOT, pipe_body, 0)
        for b in range(2):
            wait_gathers(slots[b])
        for b in range(2, NSLOT):
            wait_scatter(slots[b])
        plsc.subcore_barrier()
        pltpu.sync_copy(accum.at[pl.ds(r0, RPT)], out.at[c, pl.ds(r0, RPT)])

    return edge_kernel


@functools.lru_cache(maxsize=None)
def _get_edge_kernel(heads, hpc, chph):
    return _make_edge_kernel(heads, hpc, chph)


def _dense1_kernel(x_ref, w_ref, asd_ref, h_ref, sm_ref):
    h = jnp.dot(x_ref[...], w_ref[...], preferred_element_type=jnp.float32)
    h_ref[...] = h
    sm_ref[...] = jnp.dot(h, asd_ref[...], preferred_element_type=jnp.float32)


def _dense1(x, W1, asd1):
    BN = 1280
    nb = N // BN
    return pl.pallas_call(
        _dense1_kernel,
        grid=(nb,),
        in_specs=[
            pl.BlockSpec((BN, NUMROI), lambda i: (i, 0)),
            pl.BlockSpec((NUMROI, F), lambda i: (0, 0)),
            pl.BlockSpec((F, LN), lambda i: (0, 0)),
        ],
        out_specs=[
            pl.BlockSpec((BN, F), lambda i: (i, 0)),
            pl.BlockSpec((BN, LN), lambda i: (i, 0)),
        ],
        out_shape=[
            jax.ShapeDtypeStruct((N, F), jnp.float32),
            jax.ShapeDtypeStruct((N, LN), jnp.float32),
        ],
    )(x, W1, asd1)


def _dense2_kernel(num_ref, denw_ref, b1_ref, w2_ref, asd_ref, g_ref, sm_ref):
    t = num_ref[...] / (denw_ref[...] + 1e-16) + b1_ref[...]
    t = jnp.where(t > 0, t, jnp.exp(t) - 1.0)
    g = jnp.dot(t, w2_ref[...], preferred_element_type=jnp.float32)
    g_ref[...] = g
    sm_ref[...] = jnp.dot(g, asd_ref[...], preferred_element_type=jnp.float32)


def _dense2(num1, den1w, b1, W2, asd2):
    BN = 1280
    nb = N // BN
    return pl.pallas_call(
        _dense2_kernel,
        grid=(nb,),
        in_specs=[
            pl.BlockSpec((BN, F), lambda i: (i, 0)),
            pl.BlockSpec((BN, F), lambda i: (i, 0)),
            pl.BlockSpec((1, F), lambda i: (0, 0)),
            pl.BlockSpec((F, F), lambda i: (0, 0)),
            pl.BlockSpec((F, LN), lambda i: (0, 0)),
        ],
        out_specs=[
            pl.BlockSpec((BN, F), lambda i: (i, 0)),
            pl.BlockSpec((BN, LN), lambda i: (i, 0)),
        ],
        out_shape=[
            jax.ShapeDtypeStruct((N, F), jnp.float32),
            jax.ShapeDtypeStruct((N, LN), jnp.float32),
        ],
    )(num1, den1w, b1.reshape(1, F), W2, asd2)


def _head_kernel(num_ref, denw_ref, b2_ref, w3_ref, fc1b_ref, gamma_ref,
                 beta_ref, fc2w_ref, fc2b_ref, out_ref, acc_ref):
    k = pl.program_id(0)

    @pl.when(k == 0)
    def _():
        acc_ref[...] = jnp.zeros_like(acc_ref)

    h2 = num_ref[0] / (denw_ref[0] + 1e-16) + b2_ref[...]
    acc_ref[...] += jnp.dot(h2, w3_ref[0], preferred_element_type=jnp.float32)

    @pl.when(k == pl.num_programs(0) - 1)
    def _():
        t = acc_ref[...] + fc1b_ref[...]
        t = (t / jnp.sqrt(1.0 + 1e-5)) * gamma_ref[...] + beta_ref[...]
        t = jnp.where(t > 0, t, jnp.exp(t) - 1.0)
        out_ref[...] = jnp.dot(t, fc2w_ref[...],
                               preferred_element_type=jnp.float32) + fc2b_ref[...]


def _head(num2t, den2t, b2, fc1_w3, fc1_b, gamma, beta, fc2_w, fc2_b):
    return pl.pallas_call(
        _head_kernel,
        grid=(PER_G,),
        in_specs=[
            pl.BlockSpec((1, G, F), lambda i: (i, 0, 0)),
            pl.BlockSpec((1, G, F), lambda i: (i, 0, 0)),
            pl.BlockSpec((1, F), lambda i: (0, 0)),
            pl.BlockSpec((1, F, NUMROI), lambda i: (i, 0, 0)),
            pl.BlockSpec((1, NUMROI), lambda i: (0, 0)),
            pl.BlockSpec((1, NUMROI), lambda i: (0, 0)),
            pl.BlockSpec((1, NUMROI), lambda i: (0, 0)),
            pl.BlockSpec((NUMROI, 2), lambda i: (0, 0)),
            pl.BlockSpec((1, 2), lambda i: (0, 0)),
        ],
        out_specs=pl.BlockSpec((G, 2), lambda i: (0, 0)),
        out_shape=jax.ShapeDtypeStruct((G, 2), jnp.float32),
        scratch_shapes=[pltpu.VMEM((G, NUMROI), jnp.float32)],
    )(num2t, den2t, b2.reshape(1, F), fc1_w3, fc1_b.reshape(1, NUMROI),
      gamma.reshape(1, NUMROI), beta.reshape(1, NUMROI), fc2_w,
      fc2_b.reshape(1, 2))


def _att_matrix(a_src, a_dst, heads, out_ch):
    asf = a_src.reshape(heads * out_ch, 1)
    adf = a_dst.reshape(heads * out_ch, 1)
    hsel = (jnp.arange(heads * out_ch)[:, None] // out_ch ==
            jnp.arange(heads)[None, :]).astype(jnp.float32)
    a_s = asf * hsel
    a_d = adf * hsel
    pad = jnp.zeros((heads * out_ch, LN - 2 * heads), jnp.float32)
    return jnp.concatenate([a_s, a_d, pad], axis=1)


def kernel(x, edge_index, W1, att_src1, att_dst1, b1, W2, att_src2, att_dst2, b2,
           fc1_w, fc1_b, gamma, beta, fc2_w, fc2_b):
    src = edge_index[0]
    dst = edge_index[1]
    gsrc = jnp.stack([src, src + N])
    dperm = (dst % PER_G) * G + dst // PER_G
    zrow = jnp.zeros((N, ROW), jnp.float32)

    asd1 = _att_matrix(att_src1, att_dst1, IN_HEAD, HID)
    asd2 = _att_matrix(att_src2, att_dst2, 1, F)

    h1, sm1 = _dense1(x, W1, asd1)
    pad = jnp.zeros((N, LN), jnp.float32)
    hh1 = jnp.concatenate(
        [jnp.concatenate([h1[:, :HALF], pad], axis=1),
         jnp.concatenate([h1[:, HALF:], pad], axis=1)], axis=0)
    smx1 = jnp.concatenate([sm1, sm1], axis=0)
    acc1 = _get_edge_kernel(IN_HEAD, 2, HID)(hh1, smx1, gsrc, dst, dst, zrow)
    num1 = jnp.concatenate([acc1[0, :, :HALF], acc1[1, :, :HALF]], axis=1)
    den1 = jnp.concatenate([acc1[0, :, HALF:HALF + 2],
                            acc1[1, :, HALF:HALF + 2]], axis=1)
    den1w = jnp.repeat(den1, HID, axis=1)

    g2, sm2 = _dense2(num1, den1w, b1, W2, asd2)
    hh2 = jnp.concatenate(
        [jnp.concatenate([g2[:, :HALF], pad], axis=1),
         jnp.concatenate([g2[:, HALF:], pad], axis=1)], axis=0)
    smx2 = jnp.concatenate([sm2, sm2], axis=0)
    acc2 = _get_edge_kernel(1, 1, F)(hh2, smx2, gsrc, dst, dperm, zrow)
    num2 = jnp.concatenate([acc2[0, :, :HALF], acc2[1, :, :HALF]], axis=1)
    den2 = acc2[0, :, HALF:HALF + 1]

    num2t = num2.reshape(PER_G, G, F)
    den2t = jnp.broadcast_to(den2, (N, F)).reshape(PER_G, G, F)

    return _head(num2t, den2t, b2, fc1_w.reshape(PER_G, F, NUMROI), fc1_b,
                 gamma, beta, fc2_w, fc2_b)


# speedup vs baseline: 1.4564x; 1.4564x over previous
import functools

import jax
import jax.numpy as jnp
from jax import lax
from jax.experimental import pallas as pl
from jax.experimental.pallas import tpu as pltpu
from jax.experimental.pallas import tpu_sc as plsc

N = 10240
E = 327680
NUMROI = 128
HID = 64
IN_HEAD = 4
F = IN_HEAD * HID
G = 80
PER_G = 128

LN = 16
NSUB = 16
HALF = 128
ROW = HALF + LN
CHUNK = 40
NSLOT = 4
EPT = E // NSUB
NCHUNK = EPT // CHUNK
RPT = N // NSUB

_GDN = lax.GatherDimensionNumbers(
    offset_dims=(), collapsed_slice_dims=(0,), start_index_map=(0,))


def _lane_gather(vec, idx):
    return lax.gather(vec, idx[:, None], _GDN, (1,),
                      mode=lax.GatherScatterMode.PROMISE_IN_BOUNDS)


def _make_edge_kernel(heads, hpc, chph):
    mesh = plsc.VectorSubcoreMesh(core_axis_name="c", subcore_axis_name="s",
                                  num_cores=2, num_subcores=NSUB)

    scratch = []
    for _ in range(NSLOT):
        scratch += [
            pltpu.VMEM((3, CHUNK), jnp.int32),
            pltpu.VMEM((CHUNK, ROW), jnp.float32),
            pltpu.VMEM((CHUNK, LN), jnp.float32),
            pltpu.VMEM((CHUNK, LN), jnp.float32),
            pltpu.SemaphoreType.DMA,
            pltpu.SemaphoreType.DMA,
        ]
    scratch.append(pltpu.VMEM_SHARED((N, ROW), jnp.float32))

    @functools.partial(
        pl.kernel, mesh=mesh,
        compiler_params=pltpu.CompilerParams(use_tc_tiling_on_sc=False),
        out_type=jax.ShapeDtypeStruct((2, N, ROW), jnp.float32),
        scratch_types=scratch,
    )
    def edge_kernel(hh, smx, ipack, zrow, out, *scr):
        slots = [scr[6 * b:6 * b + 6] for b in range(NSLOT)]
        accum = scr[6 * NSLOT]
        c = lax.axis_index("c")
        s = lax.axis_index("s")
        r0 = s * RPT
        pltpu.sync_copy(zrow, accum.at[pl.ds(r0, RPT)])
        plsc.subcore_barrier()

        iota = lax.iota(jnp.int32, LN)
        lmask = iota < hpc
        hb = (c * hpc) % heads
        idx_a = jnp.where(lmask, iota + hb, 0)
        idx_b = jnp.where(lmask, iota + heads + hb, 0)
        q_lim = (s + 1) * NCHUNK - 1

        def start_chunk(gc, slot):
            idx3, hbuf, smsrc, smdst, sem_g, _ = slot
            q = jnp.minimum(s * NCHUNK + gc, q_lim)
            pltpu.sync_copy(ipack.at[c, q], idx3)
            pltpu.async_copy(hh.at[idx3.at[0]], hbuf, sem_g)
            pltpu.async_copy(smx.at[idx3.at[0]], smsrc, sem_g)
            pltpu.async_copy(smx.at[idx3.at[1]], smdst, sem_g)

        def wait_gathers(slot):
            idx3, hbuf, smsrc, smdst, sem_g, _ = slot
            pltpu.make_async_copy(hh.at[idx3.at[0]], hbuf, sem_g).wait()
            pltpu.make_async_copy(smx.at[idx3.at[0]], smsrc, sem_g).wait()
            pltpu.make_async_copy(smx.at[idx3.at[1]], smdst, sem_g).wait()

        def compute_chunk(slot):
            _, hbuf, smsrc, smdst, _, _ = slot

            def edge_body(k4, carry2):
                for u in range(4):
                    k = k4 * 4 + u
                    sv = smsrc[k]
                    dv = smdst[k]
                    e = _lane_gather(sv, idx_a) + _lane_gather(dv, idx_b)
                    e = jnp.where(lmask, e, 0.0)
                    e = jnp.where(e >= 0.0, e, 0.2 * e)
                    w = jnp.exp(e)
                    hbuf[k, pl.ds(HALF, LN)] = jnp.where(lmask, w, 0.0)
                    ws = [_lane_gather(w, jnp.full((LN,), h, jnp.int32))
                          for h in range(hpc)]
                    for j in range(HALF // LN):
                        hl = (j * LN) // chph
                        hbuf[k, pl.ds(j * LN, LN)] = (
                            hbuf[k, pl.ds(j * LN, LN)] * ws[hl])
                return carry2

            lax.fori_loop(0, CHUNK // 4, edge_body, 0)

        def issue_scatter(slot):
            idx3, hbuf, _, _, _, sem_s = slot
            pltpu.async_copy(hbuf, accum.at[idx3.at[2]], sem_s, add=True)

        def wait_scatter(slot):
            idx3, hbuf, _, _, _, sem_s = slot
            pltpu.make_async_copy(hbuf, accum.at[idx3.at[2]], sem_s).wait()

        for g in range(2):
            start_chunk(g, slots[g])

        def pipe_body(i, carry):
            for b in range(NSLOT):
                gc = i * NSLOT + b
                slot = slots[b]
                wait_gathers(slot)
                compute_chunk(slot)
                issue_scatter(slot)
                s2 = (b + 2) % NSLOT
                if b < 2:
                    @pl.when(i > 0)
                    def _():
                        wait_scatter(slots[s2])
                else:
                    wait_scatter(slots[s2])
                start_chunk(gc + 2, slots[s2])
            return carry

        lax.fori_loop(0, NCHUNK // NSLOT, pipe_body, 0)
        for b in range(2):
            wait_gathers(slots[b])
        for b in range(2, NSLOT):
            wait_scatter(slots[b])
        plsc.subcore_barrier()
        pltpu.sync_copy(accum.at[pl.ds(r0, RPT)], out.at[c, pl.ds(r0, RPT)])

    return edge_kernel


@functools.lru_cache(maxsize=None)
def _get_edge_kernel(heads, hpc, chph):
    return _make_edge_kernel(heads, hpc, chph)


def _store_hh(g_ref, h, bn):
    zpad = jnp.zeros((bn, LN), jnp.float32)
    g_ref[0, :, :] = jnp.concatenate([h[:, :HALF], zpad], axis=1)
    g_ref[1, :, :] = jnp.concatenate([h[:, HALF:], zpad], axis=1)


def _dense1_kernel(x_ref, w_ref, asd_ref, g_ref, sm_ref):
    h = jnp.dot(x_ref[...], w_ref[...], preferred_element_type=jnp.float32)
    _store_hh(g_ref, h, x_ref.shape[0])
    sm_ref[...] = jnp.dot(h, asd_ref[...], preferred_element_type=jnp.float32)


def _dense1(x, W1, asd1):
    BN = 1280
    nb = N // BN
    return pl.pallas_call(
        _dense1_kernel,
        grid=(nb,),
        in_specs=[
            pl.BlockSpec((BN, NUMROI), lambda i: (i, 0)),
            pl.BlockSpec((NUMROI, F), lambda i: (0, 0)),
            pl.BlockSpec((F, LN), lambda i: (0, 0)),
        ],
        out_specs=[
            pl.BlockSpec((2, BN, ROW), lambda i: (0, i, 0)),
            pl.BlockSpec((BN, LN), lambda i: (i, 0)),
        ],
        out_shape=[
            jax.ShapeDtypeStruct((2, N, ROW), jnp.float32),
            jax.ShapeDtypeStruct((N, LN), jnp.float32),
        ],
    )(x, W1, asd1)


def _norm_halves(a0, a1, nheads_per_core):
    parts = []
    npc = HALF // nheads_per_core
    for a in (a0, a1):
        for h in range(nheads_per_core):
            parts.append(a[:, h * npc:(h + 1) * npc] /
                         (a[:, HALF + h:HALF + h + 1] + 1e-16))
    return jnp.concatenate(parts, axis=1)


def _dense2_kernel(a0_ref, a1_ref, b1_ref, w2_ref, asd_ref, g_ref, sm_ref):
    t = _norm_halves(a0_ref[0], a1_ref[0], 2) + b1_ref[...]
    t = jnp.where(t > 0, t, jnp.exp(t) - 1.0)
    g = jnp.dot(t, w2_ref[...], preferred_element_type=jnp.float32)
    _store_hh(g_ref, g, a0_ref.shape[1])
    sm_ref[...] = jnp.dot(g, asd_ref[...], preferred_element_type=jnp.float32)


def _dense2(acc1, b1, W2, asd2):
    BN = 1280
    nb = N // BN
    return pl.pallas_call(
        _dense2_kernel,
        grid=(nb,),
        in_specs=[
            pl.BlockSpec((1, BN, ROW), lambda i: (0, i, 0)),
            pl.BlockSpec((1, BN, ROW), lambda i: (1, i, 0)),
            pl.BlockSpec((1, F), lambda i: (0, 0)),
            pl.BlockSpec((F, F), lambda i: (0, 0)),
            pl.BlockSpec((F, LN), lambda i: (0, 0)),
        ],
        out_specs=[
            pl.BlockSpec((2, BN, ROW), lambda i: (0, i, 0)),
            pl.BlockSpec((BN, LN), lambda i: (i, 0)),
        ],
        out_shape=[
            jax.ShapeDtypeStruct((2, N, ROW), jnp.float32),
            jax.ShapeDtypeStruct((N, LN), jnp.float32),
        ],
    )(acc1, acc1, b1.reshape(1, F), W2, asd2)


def _head_kernel(a0_ref, a1_ref, b2_ref, w3_ref, fc1b_ref, gamma_ref,
                 beta_ref, fc2w_ref, fc2b_ref, out_ref, acc_ref):
    k = pl.program_id(0)

    @pl.when(k == 0)
    def _():
        acc_ref[...] = jnp.zeros_like(acc_ref)

    a0 = a0_ref[0, 0]
    a1 = a1_ref[0, 0]
    h2 = jnp.concatenate(
        [a0[:, :HALF] / (a0[:, HALF:HALF + 1] + 1e-16),
         a1[:, :HALF] / (a1[:, HALF:HALF + 1] + 1e-16)], axis=1) + b2_ref[...]
    acc_ref[...] += jnp.dot(h2, w3_ref[0], preferred_element_type=jnp.float32)

    @pl.when(k == pl.num_programs(0) - 1)
    def _():
        t = acc_ref[...] + fc1b_ref[...]
        t = (t / jnp.sqrt(1.0 + 1e-5)) * gamma_ref[...] + beta_ref[...]
        t = jnp.where(t > 0, t, jnp.exp(t) - 1.0)
        out_ref[...] = jnp.dot(t, fc2w_ref[...],
                               preferred_element_type=jnp.float32) + fc2b_ref[...]


def _head(acc2r, b2, fc1_w3, fc1_b, gamma, beta, fc2_w, fc2_b):
    return pl.pallas_call(
        _head_kernel,
        grid=(PER_G,),
        in_specs=[
            pl.BlockSpec((1, 1, G, ROW), lambda i: (0, i, 0, 0)),
            pl.BlockSpec((1, 1, G, ROW), lambda i: (1, i, 0, 0)),
            pl.BlockSpec((1, F), lambda i: (0, 0)),
            pl.BlockSpec((1, F, NUMROI), lambda i: (i, 0, 0)),
            pl.BlockSpec((1, NUMROI), lambda i: (0, 0)),
            pl.BlockSpec((1, NUMROI), lambda i: (0, 0)),
            pl.BlockSpec((1, NUMROI), lambda i: (0, 0)),
            pl.BlockSpec((NUMROI, 2), lambda i: (0, 0)),
            pl.BlockSpec((1, 2), lambda i: (0, 0)),
        ],
        out_specs=pl.BlockSpec((G, 2), lambda i: (0, 0)),
        out_shape=jax.ShapeDtypeStruct((G, 2), jnp.float32),
        scratch_shapes=[pltpu.VMEM((G, NUMROI), jnp.float32)],
    )(acc2r, acc2r, b2.reshape(1, F), fc1_w3, fc1_b.reshape(1, NUMROI),
      gamma.reshape(1, NUMROI), beta.reshape(1, NUMROI), fc2_w,
      fc2_b.reshape(1, 2))


def _att_matrix(a_src, a_dst, heads, out_ch):
    asf = a_src.reshape(heads * out_ch, 1)
    adf = a_dst.reshape(heads * out_ch, 1)
    hsel = (jnp.arange(heads * out_ch)[:, None] // out_ch ==
            jnp.arange(heads)[None, :]).astype(jnp.float32)
    a_s = asf * hsel
    a_d = adf * hsel
    pad = jnp.zeros((heads * out_ch, LN - 2 * heads), jnp.float32)
    return jnp.concatenate([a_s, a_d, pad], axis=1)


def _pack_idx(gsrc, dst, sct):
    ncg = E // CHUNK
    g_r = gsrc.reshape(2, ncg, 1, CHUNK)
    d_r = jnp.broadcast_to(dst.reshape(1, ncg, 1, CHUNK), (2, ncg, 1, CHUNK))
    s_r = jnp.broadcast_to(sct.reshape(1, ncg, 1, CHUNK), (2, ncg, 1, CHUNK))
    return jnp.concatenate([g_r, d_r, s_r], axis=2)


def kernel(x, edge_index, W1, att_src1, att_dst1, b1, W2, att_src2, att_dst2, b2,
           fc1_w, fc1_b, gamma, beta, fc2_w, fc2_b):
    src = edge_index[0]
    dst = edge_index[1]
    gsrc = jnp.stack([src, src + N])
    dperm = (dst % PER_G) * G + dst // PER_G
    ipack1 = _pack_idx(gsrc, dst, dst)
    ipack2 = _pack_idx(gsrc, dst, dperm)
    zrow = jnp.zeros((RPT, ROW), jnp.float32)

    asd1 = _att_matrix(att_src1, att_dst1, IN_HEAD, HID)
    asd2 = _att_matrix(att_src2, att_dst2, 1, F)

    hh1, sm1 = _dense1(x, W1, asd1)
    smx1 = jnp.concatenate([sm1, sm1], axis=0)
    acc1 = _get_edge_kernel(IN_HEAD, 2, HID)(
        hh1.reshape(2 * N, ROW), smx1, ipack1, zrow)

    hh2, sm2 = _dense2(acc1, b1, W2, asd2)
    smx2 = jnp.concatenate([sm2, sm2], axis=0)
    acc2 = _get_edge_kernel(1, 1, F)(
        hh2.reshape(2 * N, ROW), smx2, ipack2, zrow)

    acc2r = acc2.reshape(2, PER_G, G, ROW)

    return _head(acc2r, b2, fc1_w.reshape(PER_G, F, NUMROI), fc1_b,
                 gamma, beta, fc2_w, fc2_b)


# speedup vs baseline: 1.8269x; 1.2544x over previous
import functools

import jax
import jax.numpy as jnp
from jax import lax
from jax.experimental import pallas as pl
from jax.experimental.pallas import tpu as pltpu
from jax.experimental.pallas import tpu_sc as plsc

N = 10240
E = 327680
NUMROI = 128
HID = 64
IN_HEAD = 4
F = IN_HEAD * HID
G = 80
PER_G = 128

LN = 16
NSUB = 16
HALF = 128
ROW = HALF + LN
CHUNK = 40
NSLOT = 4
EPT = E // NSUB
NCHUNK = EPT // CHUNK
RPT = N // NSUB

_GDN = lax.GatherDimensionNumbers(
    offset_dims=(), collapsed_slice_dims=(0,), start_index_map=(0,))


def _lane_gather(vec, idx):
    return lax.gather(vec, idx[:, None], _GDN, (1,),
                      mode=lax.GatherScatterMode.PROMISE_IN_BOUNDS)


def _make_edge_kernel(heads, hpc, chph):
    mesh = plsc.VectorSubcoreMesh(core_axis_name="c", subcore_axis_name="s",
                                  num_cores=2, num_subcores=NSUB)

    scratch = []
    for _ in range(NSLOT):
        scratch += [
            pltpu.VMEM((CHUNK, ROW), jnp.float32),
            pltpu.VMEM((CHUNK, LN), jnp.float32),
            pltpu.VMEM((CHUNK, LN), jnp.float32),
            pltpu.SemaphoreType.DMA,
            pltpu.SemaphoreType.DMA,
        ]
    for _ in range(2 * NSLOT):
        scratch += [
            pltpu.VMEM((3, CHUNK), jnp.int32),
            pltpu.SemaphoreType.DMA,
        ]
    scratch.append(pltpu.VMEM_SHARED((N, ROW), jnp.float32))

    @functools.partial(
        pl.kernel, mesh=mesh,
        compiler_params=pltpu.CompilerParams(use_tc_tiling_on_sc=False),
        out_type=jax.ShapeDtypeStruct((2, N, ROW), jnp.float32),
        scratch_types=scratch,
    )
    def edge_kernel(hh, smx, ipack, zrow, out, *scr):
        slots = [scr[5 * b:5 * b + 5] for b in range(NSLOT)]
        ioff = 5 * NSLOT
        islots = [scr[ioff + 2 * b:ioff + 2 * b + 2] for b in range(2 * NSLOT)]
        accum = scr[ioff + 4 * NSLOT]
        c = lax.axis_index("c")
        s = lax.axis_index("s")
        r0 = s * RPT
        pltpu.sync_copy(zrow, accum.at[pl.ds(r0, RPT)])
        plsc.subcore_barrier()

        iota = lax.iota(jnp.int32, LN)
        lmask = iota < hpc
        hb = (c * hpc) % heads
        idx_a = jnp.where(lmask, iota + hb, 0)
        idx_b = jnp.where(lmask, iota + heads + hb, 0)
        q_lim = (s + 1) * NCHUNK - 1

        def issue_idx(gc, islot):
            idx3, sem_i = islot
            q = jnp.minimum(s * NCHUNK + gc, q_lim)
            pltpu.async_copy(ipack.at[c, q], idx3, sem_i)

        def wait_idx(islot):
            idx3, sem_i = islot
            pltpu.make_async_copy(ipack.at[c, 0], idx3, sem_i).wait()

        def start_chunk(slot, islot):
            hbuf, smsrc, smdst, sem_g, _ = slot
            idx3, _ = islot
            pltpu.async_copy(hh.at[idx3.at[0]], hbuf, sem_g)
            pltpu.async_copy(smx.at[idx3.at[0]], smsrc, sem_g)
            pltpu.async_copy(smx.at[idx3.at[1]], smdst, sem_g)

        def wait_gathers(slot, islot):
            hbuf, smsrc, smdst, sem_g, _ = slot
            idx3, _ = islot
            pltpu.make_async_copy(hh.at[idx3.at[0]], hbuf, sem_g).wait()
            pltpu.make_async_copy(smx.at[idx3.at[0]], smsrc, sem_g).wait()
            pltpu.make_async_copy(smx.at[idx3.at[1]], smdst, sem_g).wait()

        def compute_chunk(slot):
            hbuf, smsrc, smdst, _, _ = slot

            def edge_body(k4, carry2):
                for u in range(4):
                    k = k4 * 4 + u
                    sv = smsrc[k]
                    dv = smdst[k]
                    e = _lane_gather(sv, idx_a) + _lane_gather(dv, idx_b)
                    e = jnp.where(lmask, e, 0.0)
                    e = jnp.where(e >= 0.0, e, 0.2 * e)
                    w = jnp.exp(e)
                    hbuf[k, pl.ds(HALF, LN)] = jnp.where(lmask, w, 0.0)
                    ws = [_lane_gather(w, jnp.full((LN,), h, jnp.int32))
                          for h in range(hpc)]
                    for j in range(HALF // LN):
                        hl = (j * LN) // chph
                        hbuf[k, pl.ds(j * LN, LN)] = (
                            hbuf[k, pl.ds(j * LN, LN)] * ws[hl])
                return carry2

            lax.fori_loop(0, CHUNK // 4, edge_body, 0)

        def issue_scatter(slot, islot):
            hbuf, _, _, _, sem_s = slot
            idx3, _ = islot
            pltpu.async_copy(hbuf, accum.at[idx3.at[2]], sem_s, add=True)

        def wait_scatter(slot, islot):
            hbuf, _, _, _, sem_s = slot
            idx3, _ = islot
            pltpu.make_async_copy(hbuf, accum.at[idx3.at[2]], sem_s).wait()

        NI = 2 * NSLOT
        for g in range(4):
            issue_idx(g, islots[g])
        for g in range(2):
            wait_idx(islots[g])
            start_chunk(slots[g], islots[g])

        def pipe_body(i, carry):
            for u in range(NI):
                gc = i * NI + u
                b = u % NSLOT
                wait_gathers(slots[b], islots[u])
                compute_chunk(slots[b])
                issue_scatter(slots[b], islots[u])
                b2 = (b + 2) % NSLOT
                u2 = (u + 2) % NI
                if u < 2:
                    @pl.when(i > 0)
                    def _():
                        wait_scatter(slots[b2], islots[u2])
                else:
                    wait_scatter(slots[b2], islots[u2])
                wait_idx(islots[u2])
                start_chunk(slots[b2], islots[u2])
                issue_idx(gc + 4, islots[(u + 4) % NI])
            return carry

        lax.fori_loop(0, NCHUNK // NI, pipe_body, 0)
        for b in range(2):
            wait_gathers(slots[b], islots[b])
        for b in range(2, NSLOT):
            wait_scatter(slots[b], islots[b])
        for u in range(2, 4):
            wait_idx(islots[u])
        plsc.subcore_barrier()
        pltpu.sync_copy(accum.at[pl.ds(r0, RPT)], out.at[c, pl.ds(r0, RPT)])

    return edge_kernel


@functools.lru_cache(maxsize=None)
def _get_edge_kernel(heads, hpc, chph):
    return _make_edge_kernel(heads, hpc, chph)


def _store_hh(g_ref, h, bn):
    zpad = jnp.zeros((bn, LN), jnp.float32)
    g_ref[0, :, :] = jnp.concatenate([h[:, :HALF], zpad], axis=1)
    g_ref[1, :, :] = jnp.concatenate([h[:, HALF:], zpad], axis=1)


def _dense1_kernel(x_ref, w_ref, asd_ref, g_ref, sm_ref):
    h = jnp.dot(x_ref[...], w_ref[...], preferred_element_type=jnp.float32)
    _store_hh(g_ref, h, x_ref.shape[0])
    sm_ref[...] = jnp.dot(h, asd_ref[...], preferred_element_type=jnp.float32)


def _dense1(x, W1, asd1):
    BN = 1280
    nb = N // BN
    return pl.pallas_call(
        _dense1_kernel,
        grid=(nb,),
        in_specs=[
            pl.BlockSpec((BN, NUMROI), lambda i: (i, 0)),
            pl.BlockSpec((NUMROI, F), lambda i: (0, 0)),
            pl.BlockSpec((F, LN), lambda i: (0, 0)),
        ],
        out_specs=[
            pl.BlockSpec((2, BN, ROW), lambda i: (0, i, 0)),
            pl.BlockSpec((BN, LN), lambda i: (i, 0)),
        ],
        out_shape=[
            jax.ShapeDtypeStruct((2, N, ROW), jnp.float32),
            jax.ShapeDtypeStruct((N, LN), jnp.float32),
        ],
    )(x, W1, asd1)


def _norm_halves(a0, a1, nheads_per_core):
    parts = []
    npc = HALF // nheads_per_core
    for a in (a0, a1):
        for h in range(nheads_per_core):
            parts.append(a[:, h * npc:(h + 1) * npc] /
                         (a[:, HALF + h:HALF + h + 1] + 1e-16))
    return jnp.concatenate(parts, axis=1)


def _dense2_kernel(a0_ref, a1_ref, b1_ref, w2_ref, asd_ref, g_ref, sm_ref):
    t = _norm_halves(a0_ref[0], a1_ref[0], 2) + b1_ref[...]
    t = jnp.where(t > 0, t, jnp.exp(t) - 1.0)
    g = jnp.dot(t, w2_ref[...], preferred_element_type=jnp.float32)
    _store_hh(g_ref, g, a0_ref.shape[1])
    sm_ref[...] = jnp.dot(g, asd_ref[...], preferred_element_type=jnp.float32)


def _dense2(acc1, b1, W2, asd2):
    BN = 1280
    nb = N // BN
    return pl.pallas_call(
        _dense2_kernel,
        grid=(nb,),
        in_specs=[
            pl.BlockSpec((1, BN, ROW), lambda i: (0, i, 0)),
            pl.BlockSpec((1, BN, ROW), lambda i: (1, i, 0)),
            pl.BlockSpec((1, F), lambda i: (0, 0)),
            pl.BlockSpec((F, F), lambda i: (0, 0)),
            pl.BlockSpec((F, LN), lambda i: (0, 0)),
        ],
        out_specs=[
            pl.BlockSpec((2, BN, ROW), lambda i: (0, i, 0)),
            pl.BlockSpec((BN, LN), lambda i: (i, 0)),
        ],
        out_shape=[
            jax.ShapeDtypeStruct((2, N, ROW), jnp.float32),
            jax.ShapeDtypeStruct((N, LN), jnp.float32),
        ],
    )(acc1, acc1, b1.reshape(1, F), W2, asd2)


def _head_kernel(a0_ref, a1_ref, b2_ref, w3_ref, fc1b_ref, gamma_ref,
                 beta_ref, fc2w_ref, fc2b_ref, out_ref, acc_ref):
    k = pl.program_id(0)

    @pl.when(k == 0)
    def _():
        acc_ref[...] = jnp.zeros_like(acc_ref)

    a0 = a0_ref[0, 0]
    a1 = a1_ref[0, 0]
    h2 = jnp.concatenate(
        [a0[:, :HALF] / (a0[:, HALF:HALF + 1] + 1e-16),
         a1[:, :HALF] / (a1[:, HALF:HALF + 1] + 1e-16)], axis=1) + b2_ref[...]
    acc_ref[...] += jnp.dot(h2, w3_ref[0], preferred_element_type=jnp.float32)

    @pl.when(k == pl.num_programs(0) - 1)
    def _():
        t = acc_ref[...] + fc1b_ref[...]
        t = (t / jnp.sqrt(1.0 + 1e-5)) * gamma_ref[...] + beta_ref[...]
        t = jnp.where(t > 0, t, jnp.exp(t) - 1.0)
        out_ref[...] = jnp.dot(t, fc2w_ref[...],
                               preferred_element_type=jnp.float32) + fc2b_ref[...]


def _head(acc2r, b2, fc1_w3, fc1_b, gamma, beta, fc2_w, fc2_b):
    return pl.pallas_call(
        _head_kernel,
        grid=(PER_G,),
        in_specs=[
            pl.BlockSpec((1, 1, G, ROW), lambda i: (0, i, 0, 0)),
            pl.BlockSpec((1, 1, G, ROW), lambda i: (1, i, 0, 0)),
            pl.BlockSpec((1, F), lambda i: (0, 0)),
            pl.BlockSpec((1, F, NUMROI), lambda i: (i, 0, 0)),
            pl.BlockSpec((1, NUMROI), lambda i: (0, 0)),
            pl.BlockSpec((1, NUMROI), lambda i: (0, 0)),
            pl.BlockSpec((1, NUMROI), lambda i: (0, 0)),
            pl.BlockSpec((NUMROI, 2), lambda i: (0, 0)),
            pl.BlockSpec((1, 2), lambda i: (0, 0)),
        ],
        out_specs=pl.BlockSpec((G, 2), lambda i: (0, 0)),
        out_shape=jax.ShapeDtypeStruct((G, 2), jnp.float32),
        scratch_shapes=[pltpu.VMEM((G, NUMROI), jnp.float32)],
    )(acc2r, acc2r, b2.reshape(1, F), fc1_w3, fc1_b.reshape(1, NUMROI),
      gamma.reshape(1, NUMROI), beta.reshape(1, NUMROI), fc2_w,
      fc2_b.reshape(1, 2))


def _att_matrix(a_src, a_dst, heads, out_ch):
    asf = a_src.reshape(heads * out_ch, 1)
    adf = a_dst.reshape(heads * out_ch, 1)
    hsel = (jnp.arange(heads * out_ch)[:, None] // out_ch ==
            jnp.arange(heads)[None, :]).astype(jnp.float32)
    a_s = asf * hsel
    a_d = adf * hsel
    pad = jnp.zeros((heads * out_ch, LN - 2 * heads), jnp.float32)
    return jnp.concatenate([a_s, a_d, pad], axis=1)


def _pack_idx(gsrc, dst, sct):
    ncg = E // CHUNK
    g_r = gsrc.reshape(2, ncg, 1, CHUNK)
    d_r = jnp.broadcast_to(dst.reshape(1, ncg, 1, CHUNK), (2, ncg, 1, CHUNK))
    s_r = jnp.broadcast_to(sct.reshape(1, ncg, 1, CHUNK), (2, ncg, 1, CHUNK))
    return jnp.concatenate([g_r, d_r, s_r], axis=2)


def kernel(x, edge_index, W1, att_src1, att_dst1, b1, W2, att_src2, att_dst2, b2,
           fc1_w, fc1_b, gamma, beta, fc2_w, fc2_b):
    src = edge_index[0]
    dst = edge_index[1]
    gsrc = jnp.stack([src, src + N])
    dperm = (dst % PER_G) * G + dst // PER_G
    ipack1 = _pack_idx(gsrc, dst, dst)
    ipack2 = _pack_idx(gsrc, dst, dperm)
    zrow = jnp.zeros((RPT, ROW), jnp.float32)

    asd1 = _att_matrix(att_src1, att_dst1, IN_HEAD, HID)
    asd2 = _att_matrix(att_src2, att_dst2, 1, F)

    hh1, sm1 = _dense1(x, W1, asd1)
    smx1 = jnp.concatenate([sm1, sm1], axis=0)
    acc1 = _get_edge_kernel(IN_HEAD, 2, HID)(
        hh1.reshape(2 * N, ROW), smx1, ipack1, zrow)

    hh2, sm2 = _dense2(acc1, b1, W2, asd2)
    smx2 = jnp.concatenate([sm2, sm2], axis=0)
    acc2 = _get_edge_kernel(1, 1, F)(
        hh2.reshape(2 * N, ROW), smx2, ipack2, zrow)

    acc2r = acc2.reshape(2, PER_G, G, ROW)

    return _head(acc2r, b2, fc1_w.reshape(PER_G, F, NUMROI), fc1_b,
                 gamma, beta, fc2_w, fc2_b)
